# baseline (speedup 1.0000x reference)
import functools

import numpy as np
import jax
import jax.numpy as jnp
from jax import lax
from jax.experimental import pallas as pl
from jax.experimental.pallas import tpu as pltpu
from jax.experimental.pallas import tpu_sc as plsc

_B = 128
_N = 32768
_RANK = int(np.int32(0.9 * np.float32(_N)))
_NC = 2
_NS = 16
_NW = _NC * _NS
_RPW = _B // _NW
_MIN32 = np.int32(-2147483648)
_CHUNKS = _N // 16
_U = 8
_NH = 4
_NBINS = 256
_CSEG = 2048


def _scalar(v):
    return jax.lax.index_in_dim(v, 0, keepdims=False)


def _sc_body(x_hbm, th_hbm, xa, xb, cand, h0, h1, h2, h3, hist, tbuf, sem):
    wid = lax.axis_index("s") * _NC + lax.axis_index("c")
    lanes = lax.iota(jnp.int32, 16)
    zeros16 = jnp.broadcast_to(np.int32(0), (16,))
    ones16 = jnp.broadcast_to(np.int32(1), (16,))
    loff8 = lanes * np.int32(_NBINS)
    loffc = lanes * np.int32(_CSEG)
    hbufs = [h0, h1, h2, h3]
    xbufs = [xa, xb]
    row0 = wid * _RPW

    def clear_hists(nh):
        def st(c, carry):
            for k in range(_U):
                base = (c * _U + k) * 16
                for hb in hbufs[:nh]:
                    hb[pl.ds(base, 16)] = zeros16
            return carry
        lax.fori_loop(0, _NBINS // _U, st, 0)

    def find(nh, r):
        def mstep(c, tv):
            acc = hbufs[0][pl.ds(c * 16, 16)]
            for hb in hbufs[:nh]:
                for l in range(16):
                    if hb is hbufs[0] and l == 0:
                        continue
                    acc = acc + hb[pl.ds(l * _NBINS + c * 16, 16)]
            hist[pl.ds(c * 16, 16)] = acc
            return jnp.where(lanes == c, jnp.sum(acc), tv)

        totals = lax.fori_loop(0, _NBINS // 16, mstep, zeros16)
        cum = plsc.cumsum(totals)
        pred = cum > r
        cidx = _scalar(plsc.all_reduce_ffs(pred))
        base_before = jnp.sum(jnp.where(lanes < cidx, totals, 0))
        rr = r - base_before
        h = hist[pl.ds(cidx * 16, 16)]
        cum2 = plsc.cumsum(h)
        pred2 = cum2 > rr
        ffs = _scalar(plsc.all_reduce_ffs(pred2))
        cnt_before = jnp.sum(jnp.where(lanes < ffs, h, 0))
        return cidx * 16 + ffs, rr - cnt_before

    pltpu.async_copy(x_hbm.at[pl.ds(row0 * _N, _N)], xa, sem)

    tvec = zeros16
    for r_i in range(_RPW):
        xv = xbufs[r_i % 2]
        xnext = xbufs[(r_i + 1) % 2]
        rowbase = (row0 + r_i) * _N
        pltpu.make_async_copy(x_hbm.at[pl.ds(rowbase, _N)], xv, sem).wait()

        clear_hists(_NH)

        @plsc.parallel_loop(0, _CHUNKS, _NH, unroll=2)
        def pass_a(c, xv=xv):
            for k in range(_NH):
                j = c + k
                x = xv[pl.ds(j * 16, 16)]
                u = lax.bitcast_convert_type(x, jnp.int32)
                s = u ^ (lax.shift_right_arithmetic(u, 31) | _MIN32)
                b = lax.shift_right_logical(s, 24) | loff8
                plsc.addupdate_scatter(hbufs[k], [b], ones16)
        v0, r1 = find(_NH, np.int32(_RANK))

        def pass_b(c, cntv, xv=xv):
            for k in range(_NH):
                j = c + k
                x = xv[pl.ds(j * 16, 16)]
                u = lax.bitcast_convert_type(x, jnp.int32)
                s = u ^ (lax.shift_right_arithmetic(u, 31) | _MIN32)
                pm = lax.shift_right_logical(s, 24) == v0
                plsc.store_scatter(cand, [loffc | cntv], s, mask=pm)
                cntv = cntv + pm.astype(jnp.int32)
            return cntv

        cntv = plsc.parallel_loop(
            0, _CHUNKS, _NH, unroll=2, carry=zeros16)(pass_b)

        if r_i + 1 < _RPW:
            pltpu.async_copy(x_hbm.at[pl.ds(rowbase + _N, _N)], xnext, sem)

        clear_hists(2)
        nmax0 = jnp.max(cntv)

        def pass_h1(c):
            for k in range(2):
                j = c + k
                sg = plsc.load_gather(cand, [loffc + j])
                pm = cntv > j
                b = (lax.shift_right_logical(sg, 16) & np.int32(0xFF)) | loff8
                plsc.addupdate_scatter(hbufs[k], [b], ones16, mask=pm)

        plsc.parallel_loop(0, ((nmax0 + 1) // 2) * 2, 2, unroll=1)(pass_h1)
        v1, r2 = find(2, r1)
        pref16 = (v0 << 8) | v1

        clear_hists(2)
        nmax = jnp.max(cntv)

        def pass_c(c, cntv2):
            for k in range(2):
                j = c + k
                sg = plsc.load_gather(cand, [loffc + j])
                pm = (cntv > j) & (lax.shift_right_logical(sg, 16) == pref16)
                plsc.store_scatter(cand, [loffc | cntv2], sg, mask=pm)
                cntv2 = cntv2 + pm.astype(jnp.int32)
                b = (lax.shift_right_logical(sg, 8) & np.int32(0xFF)) | loff8
                plsc.addupdate_scatter(hbufs[k], [b], ones16, mask=pm)
            return cntv2

        cntv2 = plsc.parallel_loop(
            0, ((nmax + 1) // 2) * 2, 2, unroll=1, carry=zeros16)(pass_c)
        v2, r3 = find(2, r2)
        pref24 = (pref16 << 8) | v2

        clear_hists(2)
        nmax2 = jnp.max(cntv2)

        @plsc.parallel_loop(0, ((nmax2 + 1) // 2) * 2, 2, unroll=1)
        def pass_d(c):
            for k in range(2):
                j = c + k
                sg = plsc.load_gather(cand, [loffc + j])
                pm = (cntv2 > j) & (lax.shift_right_logical(sg, 8) == pref24)
                b = (sg & np.int32(0xFF)) | loff8
                plsc.addupdate_scatter(hbufs[k], [b], ones16, mask=pm)
        v3, _ = find(2, r3)

        theta_s = (v0 << 24) | (v1 << 16) | (v2 << 8) | v3
        tvec = jnp.where(lanes == r_i, theta_s, tvec)

    tbuf[...] = tvec
    pltpu.sync_copy(tbuf, th_hbm.at[wid])


@functools.cache
def _build_sc():
    mesh = plsc.VectorSubcoreMesh(
        core_axis_name="c", subcore_axis_name="s", num_cores=_NC)
    return pl.kernel(
        _sc_body,
        out_type=jax.ShapeDtypeStruct((_NW, 16), jnp.int32),
        mesh=mesh,
        compiler_params=pltpu.CompilerParams(needs_layout_passes=False),
        scratch_types=[
            pltpu.VMEM((_N,), jnp.float32),
            pltpu.VMEM((_N,), jnp.float32),
            pltpu.VMEM((_N + 16,), jnp.int32),
            pltpu.VMEM((16 * _NBINS,), jnp.int32),
            pltpu.VMEM((16 * _NBINS,), jnp.int32),
            pltpu.VMEM((16 * _NBINS,), jnp.int32),
            pltpu.VMEM((16 * _NBINS,), jnp.int32),
            pltpu.VMEM((_NBINS,), jnp.int32),
            pltpu.VMEM((16,), jnp.int32),
            pltpu.SemaphoreType.DMA,
        ],
    )


def _tc_mask_body(t_ref, x_ref, o_ref):
    ts = t_ref[...]
    u = ts ^ (jnp.bitwise_not(lax.shift_right_arithmetic(ts, 31)) | _MIN32)
    tf = lax.bitcast_convert_type(u, jnp.float32)
    x = x_ref[...]
    o_ref[...] = jnp.where(x >= tf, x, np.float32(0.0))


_TC_BLK = 4096


@functools.cache
def _build_tc():
    return pl.pallas_call(
        _tc_mask_body,
        grid=(_N // _TC_BLK,),
        in_specs=[
            pl.BlockSpec((_B, 1), lambda i: (0, 0)),
            pl.BlockSpec((_B, _TC_BLK), lambda i: (0, i)),
        ],
        out_specs=pl.BlockSpec((_B, _TC_BLK), lambda i: (0, i)),
        out_shape=jax.ShapeDtypeStruct((_B, _N), jnp.float32),
    )


@jax.jit
def kernel(X):
    ts = _build_sc()(X.reshape(_B * _N))
    th = ts[:, :_RPW].reshape(_B, 1)
    return _build_tc()(th, X)


# speedup vs baseline: 1.0442x; 1.0442x over previous
import functools

import numpy as np
import jax
import jax.numpy as jnp
from jax import lax
from jax.experimental import pallas as pl
from jax.experimental.pallas import tpu as pltpu
from jax.experimental.pallas import tpu_sc as plsc

_B = 128
_N = 32768
_RANK = int(np.int32(0.9 * np.float32(_N)))
_NC = 2
_NS = 16
_NW = _NC * _NS
_RPW = _B // _NW
_MIN32 = np.int32(-2147483648)
_CHUNKS = _N // 16
_U = 8
_NH = 4
_NBINS = 256
_CSEG = 2048


def _scalar(v):
    return jax.lax.index_in_dim(v, 0, keepdims=False)


def _sc_body(x_hbm, th_hbm, xa, xb, cand, h0, h1, h2, h3, hist, tbuf, sem):
    wid = lax.axis_index("s") * _NC + lax.axis_index("c")
    lanes = lax.iota(jnp.int32, 16)
    zeros16 = jnp.broadcast_to(np.int32(0), (16,))
    ones16 = jnp.broadcast_to(np.int32(1), (16,))
    loff8 = lanes * np.int32(_NBINS)
    loffc = lanes * np.int32(_CSEG)
    hbufs = [h0, h1, h2, h3]
    xbufs = [xa, xb]
    row0 = wid * _RPW

    def clear_hists(nh):
        def st(c, carry):
            for k in range(_U):
                base = (c * _U + k) * 16
                for hb in hbufs[:nh]:
                    hb[pl.ds(base, 16)] = zeros16
            return carry
        lax.fori_loop(0, _NBINS // _U, st, 0)

    def find(nh, r):
        def mstep(c, tv):
            acc = hbufs[0][pl.ds(c * 16, 16)]
            for hb in hbufs[:nh]:
                for l in range(16):
                    if hb is hbufs[0] and l == 0:
                        continue
                    acc = acc + hb[pl.ds(l * _NBINS + c * 16, 16)]
            hist[pl.ds(c * 16, 16)] = acc
            return jnp.where(lanes == c, jnp.sum(acc), tv)

        totals = lax.fori_loop(0, _NBINS // 16, mstep, zeros16)
        cum = plsc.cumsum(totals)
        pred = cum > r
        cidx = _scalar(plsc.all_reduce_ffs(pred))
        base_before = jnp.sum(jnp.where(lanes < cidx, totals, 0))
        rr = r - base_before
        h = hist[pl.ds(cidx * 16, 16)]
        cum2 = plsc.cumsum(h)
        pred2 = cum2 > rr
        ffs = _scalar(plsc.all_reduce_ffs(pred2))
        cnt_before = jnp.sum(jnp.where(lanes < ffs, h, 0))
        return cidx * 16 + ffs, rr - cnt_before

    pltpu.async_copy(x_hbm.at[pl.ds(row0 * _N, _N)], xa, sem)

    tvec = zeros16
    for r_i in range(_RPW):
        xv = xbufs[r_i % 2]
        xnext = xbufs[(r_i + 1) % 2]
        rowbase = (row0 + r_i) * _N
        pltpu.make_async_copy(x_hbm.at[pl.ds(rowbase, _N)], xv, sem).wait()

        clear_hists(_NH)

        @plsc.parallel_loop(0, _CHUNKS, _NH, unroll=1)
        def pass_a(c, xv=xv):
            for k in range(_NH):
                j = c + k
                x = xv[pl.ds(j * 16, 16)]
                u = lax.bitcast_convert_type(x, jnp.int32)
                s = u ^ (lax.shift_right_arithmetic(u, 31) | _MIN32)
                b = lax.shift_right_logical(s, 24) | loff8
                plsc.addupdate_scatter(hbufs[k], [b], ones16)
        v0, r1 = find(_NH, np.int32(_RANK))

        def pass_b(c, cntv, xv=xv):
            for k in range(_NH):
                j = c + k
                x = xv[pl.ds(j * 16, 16)]
                u = lax.bitcast_convert_type(x, jnp.int32)
                s = u ^ (lax.shift_right_arithmetic(u, 31) | _MIN32)
                pm = lax.shift_right_logical(s, 24) == v0
                plsc.store_scatter(cand, [loffc | cntv], s, mask=pm)
                cntv = cntv + pm.astype(jnp.int32)
            return cntv

        cntv = plsc.parallel_loop(
            0, _CHUNKS, _NH, unroll=1, carry=zeros16)(pass_b)

        if r_i + 1 < _RPW:
            pltpu.async_copy(x_hbm.at[pl.ds(rowbase + _N, _N)], xnext, sem)

        clear_hists(2)
        nmax0 = jnp.max(cntv)

        def pass_h1(c):
            for k in range(2):
                j = c + k
                sg = plsc.load_gather(cand, [loffc + j])
                pm = cntv > j
                b = (lax.shift_right_logical(sg, 16) & np.int32(0xFF)) | loff8
                plsc.addupdate_scatter(hbufs[k], [b], ones16, mask=pm)

        plsc.parallel_loop(0, ((nmax0 + 1) // 2) * 2, 2, unroll=1)(pass_h1)
        v1, r2 = find(2, r1)
        pref16 = (v0 << 8) | v1

        clear_hists(2)
        nmax = jnp.max(cntv)

        def pass_c(c, cntv2):
            for k in range(2):
                j = c + k
                sg = plsc.load_gather(cand, [loffc + j])
                pm = (cntv > j) & (lax.shift_right_logical(sg, 16) == pref16)
                plsc.store_scatter(cand, [loffc | cntv2], sg, mask=pm)
                cntv2 = cntv2 + pm.astype(jnp.int32)
                b = (lax.shift_right_logical(sg, 8) & np.int32(0xFF)) | loff8
                plsc.addupdate_scatter(hbufs[k], [b], ones16, mask=pm)
            return cntv2

        cntv2 = plsc.parallel_loop(
            0, ((nmax + 1) // 2) * 2, 2, unroll=1, carry=zeros16)(pass_c)
        v2, r3 = find(2, r2)
        pref24 = (pref16 << 8) | v2

        clear_hists(2)
        nmax2 = jnp.max(cntv2)

        @plsc.parallel_loop(0, ((nmax2 + 1) // 2) * 2, 2, unroll=1)
        def pass_d(c):
            for k in range(2):
                j = c + k
                sg = plsc.load_gather(cand, [loffc + j])
                pm = (cntv2 > j) & (lax.shift_right_logical(sg, 8) == pref24)
                b = (sg & np.int32(0xFF)) | loff8
                plsc.addupdate_scatter(hbufs[k], [b], ones16, mask=pm)
        v3, _ = find(2, r3)

        theta_s = (v0 << 24) | (v1 << 16) | (v2 << 8) | v3
        tvec = jnp.where(lanes == r_i, theta_s, tvec)

    tbuf[...] = tvec
    pltpu.sync_copy(tbuf, th_hbm.at[wid])


@functools.cache
def _build_sc():
    mesh = plsc.VectorSubcoreMesh(
        core_axis_name="c", subcore_axis_name="s", num_cores=_NC)
    return pl.kernel(
        _sc_body,
        out_type=jax.ShapeDtypeStruct((_NW, 16), jnp.int32),
        mesh=mesh,
        compiler_params=pltpu.CompilerParams(needs_layout_passes=False),
        scratch_types=[
            pltpu.VMEM((_N,), jnp.float32),
            pltpu.VMEM((_N,), jnp.float32),
            pltpu.VMEM((_N + 16,), jnp.int32),
            pltpu.VMEM((16 * _NBINS,), jnp.int32),
            pltpu.VMEM((16 * _NBINS,), jnp.int32),
            pltpu.VMEM((16 * _NBINS,), jnp.int32),
            pltpu.VMEM((16 * _NBINS,), jnp.int32),
            pltpu.VMEM((_NBINS,), jnp.int32),
            pltpu.VMEM((16,), jnp.int32),
            pltpu.SemaphoreType.DMA,
        ],
    )


def _tc_mask_body(t_ref, x_ref, o_ref):
    ts = t_ref[...]
    u = ts ^ (jnp.bitwise_not(lax.shift_right_arithmetic(ts, 31)) | _MIN32)
    tf = lax.bitcast_convert_type(u, jnp.float32)
    x = x_ref[...]
    o_ref[...] = jnp.where(x >= tf, x, np.float32(0.0))


_TC_BLK = 4096


@functools.cache
def _build_tc():
    return pl.pallas_call(
        _tc_mask_body,
        grid=(_N // _TC_BLK,),
        in_specs=[
            pl.BlockSpec((_B, 1), lambda i: (0, 0)),
            pl.BlockSpec((_B, _TC_BLK), lambda i: (0, i)),
        ],
        out_specs=pl.BlockSpec((_B, _TC_BLK), lambda i: (0, i)),
        out_shape=jax.ShapeDtypeStruct((_B, _N), jnp.float32),
    )


@jax.jit
def kernel(X):
    ts = _build_sc()(X.reshape(_B * _N))
    th = ts[:, :_RPW].reshape(_B, 1)
    return _build_tc()(th, X)


# speedup vs baseline: 1.3685x; 1.3106x over previous
import functools

import numpy as np
import jax
import jax.numpy as jnp
from jax import lax
from jax.experimental import pallas as pl
from jax.experimental.pallas import tpu as pltpu
from jax.experimental.pallas import tpu_sc as plsc

_B = 128
_N = 32768
_RANK = int(np.int32(0.9 * np.float32(_N)))
_NC = 2
_NS = 16
_NW = _NC * _NS
_RPW = _B // _NW
_MIN32 = np.int32(-2147483648)
_CHUNKS = _N // 16
_U = 8
_NH = 4
_NBINS = 256
_CSEG = 2048


def _scalar(v):
    return jax.lax.index_in_dim(v, 0, keepdims=False)


def _sc_body(x_hbm, th_hbm, xa, xb, cand, h0, h1, h2, h3, hist, tbuf, sem):
    wid = lax.axis_index("s") * _NC + lax.axis_index("c")
    lanes = lax.iota(jnp.int32, 16)
    zeros16 = jnp.broadcast_to(np.int32(0), (16,))
    ones16 = jnp.broadcast_to(np.int32(1), (16,))
    loff8 = lanes * np.int32(_NBINS)
    loffc = lanes * np.int32(_CSEG)
    hbufs = [h0, h1, h2, h3]
    xbufs = [xa, xb]
    row0 = wid * _RPW

    def clear_hists(nh):
        def st(c, carry):
            for k in range(_U):
                base = (c * _U + k) * 16
                for hb in hbufs[:nh]:
                    hb[pl.ds(base, 16)] = zeros16
            return carry
        lax.fori_loop(0, _NBINS // _U, st, 0)

    def find(nh, r):
        def mstep(c, tv):
            acc = hbufs[0][pl.ds(c * 16, 16)]
            for hb in hbufs[:nh]:
                for l in range(16):
                    if hb is hbufs[0] and l == 0:
                        continue
                    acc = acc + hb[pl.ds(l * _NBINS + c * 16, 16)]
            hist[pl.ds(c * 16, 16)] = acc
            return jnp.where(lanes == c, jnp.sum(acc), tv)

        totals = lax.fori_loop(0, _NBINS // 16, mstep, zeros16)
        cum = plsc.cumsum(totals)
        pred = cum > r
        cidx = _scalar(plsc.all_reduce_ffs(pred))
        base_before = jnp.sum(jnp.where(lanes < cidx, totals, 0))
        rr = r - base_before
        h = hist[pl.ds(cidx * 16, 16)]
        cum2 = plsc.cumsum(h)
        pred2 = cum2 > rr
        ffs = _scalar(plsc.all_reduce_ffs(pred2))
        cnt_before = jnp.sum(jnp.where(lanes < ffs, h, 0))
        return cidx * 16 + ffs, rr - cnt_before

    pltpu.async_copy(x_hbm.at[row0], xa, sem)

    for r_i in range(_RPW):
        xv = xbufs[r_i % 2]
        xnext = xbufs[(r_i + 1) % 2]
        pltpu.make_async_copy(x_hbm.at[row0 + r_i], xv, sem).wait()

        clear_hists(2)

        @plsc.parallel_loop(0, _CHUNKS, 2, unroll=1)
        def pass_a(c, xv=xv):
            for k in range(2):
                j = c + k
                x = xv[pl.ds(j * 16, 16)]
                u = lax.bitcast_convert_type(x, jnp.int32)
                s = u ^ (lax.shift_right_arithmetic(u, 31) | _MIN32)
                b = lax.shift_right_logical(s, 24) | loff8
                plsc.addupdate_scatter(hbufs[k], [b], ones16)
        v0, r1 = find(2, np.int32(_RANK))

        clear_hists(2)

        def pass_b(c, cntv, xv=xv):
            for k in range(2):
                j = c + k
                x = xv[pl.ds(j * 16, 16)]
                u = lax.bitcast_convert_type(x, jnp.int32)
                s = u ^ (lax.shift_right_arithmetic(u, 31) | _MIN32)
                pm = lax.shift_right_logical(s, 24) == v0
                plsc.store_scatter(cand, [loffc | cntv], s, mask=pm)
                cntv = jnp.where(pm, cntv + ones16, cntv)
                b = (lax.shift_right_logical(s, 16) & np.int32(0xFF)) | loff8
                plsc.addupdate_scatter(hbufs[k], [b], ones16, mask=pm)
            return cntv

        cntv = plsc.parallel_loop(
            0, _CHUNKS, 2, unroll=1, carry=zeros16)(pass_b)

        if r_i + 1 < _RPW:
            pltpu.async_copy(x_hbm.at[row0 + r_i + 1], xnext, sem)

        v1, r2 = find(2, r1)
        pref16 = (v0 << 8) | v1

        clear_hists(2)
        nmax = jnp.max(cntv)

        def pass_c(c, cntv2):
            for k in range(2):
                j = c + k
                sg = plsc.load_gather(cand, [loffc + j])
                pm = (cntv > j) & (lax.shift_right_logical(sg, 16) == pref16)
                plsc.store_scatter(cand, [loffc | cntv2], sg, mask=pm)
                cntv2 = cntv2 + pm.astype(jnp.int32)
                b = (lax.shift_right_logical(sg, 8) & np.int32(0xFF)) | loff8
                plsc.addupdate_scatter(hbufs[k], [b], ones16, mask=pm)
            return cntv2

        cntv2 = plsc.parallel_loop(
            0, ((nmax + 1) // 2) * 2, 2, unroll=1, carry=zeros16)(pass_c)
        v2, r3 = find(2, r2)
        pref24 = (pref16 << 8) | v2

        clear_hists(2)
        nmax2 = jnp.max(cntv2)

        @plsc.parallel_loop(0, ((nmax2 + 1) // 2) * 2, 2, unroll=1)
        def pass_d(c):
            for k in range(2):
                j = c + k
                sg = plsc.load_gather(cand, [loffc + j])
                pm = (cntv2 > j) & (lax.shift_right_logical(sg, 8) == pref24)
                b = (sg & np.int32(0xFF)) | loff8
                plsc.addupdate_scatter(hbufs[k], [b], ones16, mask=pm)
        v3, _ = find(2, r3)

        theta_s = (v0 << 24) | (v1 << 16) | (v2 << 8) | v3
        tbuf[...] = jnp.broadcast_to(theta_s, (16,))
        pltpu.sync_copy(tbuf, th_hbm.at[row0 + r_i])


@functools.cache
def _build_sc():
    mesh = plsc.VectorSubcoreMesh(
        core_axis_name="c", subcore_axis_name="s", num_cores=_NC)
    return pl.kernel(
        _sc_body,
        out_type=jax.ShapeDtypeStruct((_B, 16), jnp.int32),
        mesh=mesh,
        compiler_params=pltpu.CompilerParams(needs_layout_passes=False),
        scratch_types=[
            pltpu.VMEM((_N,), jnp.float32),
            pltpu.VMEM((_N,), jnp.float32),
            pltpu.VMEM((_N + 16,), jnp.int32),
            pltpu.VMEM((16 * _NBINS,), jnp.int32),
            pltpu.VMEM((16 * _NBINS,), jnp.int32),
            pltpu.VMEM((16 * _NBINS,), jnp.int32),
            pltpu.VMEM((16 * _NBINS,), jnp.int32),
            pltpu.VMEM((_NBINS,), jnp.int32),
            pltpu.VMEM((16,), jnp.int32),
            pltpu.SemaphoreType.DMA,
        ],
    )


def _tc_mask_body(t_ref, x_ref, o_ref):
    ts = t_ref[:, 0:1]
    u = ts ^ (jnp.bitwise_not(lax.shift_right_arithmetic(ts, 31)) | _MIN32)
    tf = lax.bitcast_convert_type(u, jnp.float32)
    x = x_ref[...]
    o_ref[...] = jnp.where(x >= tf, x, np.float32(0.0))


_TC_BLK = 4096


@functools.cache
def _build_tc():
    return pl.pallas_call(
        _tc_mask_body,
        grid=(_N // _TC_BLK,),
        in_specs=[
            pl.BlockSpec((_B, 16), lambda i: (0, 0)),
            pl.BlockSpec((_B, _TC_BLK), lambda i: (0, i)),
        ],
        out_specs=pl.BlockSpec((_B, _TC_BLK), lambda i: (0, i)),
        out_shape=jax.ShapeDtypeStruct((_B, _N), jnp.float32),
    )


@jax.jit
def kernel(X):
    ts = _build_sc()(X)
    return _build_tc()(ts, X)
